# baseline (speedup 1.0000x reference)
import functools

import jax
import jax.numpy as jnp
from jax import lax
from jax.experimental import pallas as pl
from jax.experimental.pallas import tpu as pltpu
from jax.experimental.pallas import tpu_sc as plsc

VOCAB_SIZE = 1000000
EMBED = 64
BATCH = 4096
SEQ = 50
N = BATCH * SEQ
NC = 2
NS = 16
NW = NC * NS
CHUNK = 128
PER_W = N // NW
NCH = PER_W // CHUNK
NBUF = 5

HALF_V = VOCAB_SIZE // 2
DEPAD_ROWS = 5000

_mesh = plsc.VectorSubcoreMesh(core_axis_name="c", subcore_axis_name="s")


def _repack_body(lo_ref, hi_ref, out_ref):
    out_ref[...] = jnp.concatenate([lo_ref[...], hi_ref[...]], axis=1)


_repack = pl.pallas_call(
    _repack_body,
    grid=(HALF_V // DEPAD_ROWS,),
    in_specs=[
        pl.BlockSpec((DEPAD_ROWS, EMBED), lambda g: (g, 0)),
        pl.BlockSpec((DEPAD_ROWS, EMBED), lambda g: (g + HALF_V // DEPAD_ROWS, 0)),
    ],
    out_specs=pl.BlockSpec((DEPAD_ROWS, 2 * EMBED), lambda g: (g, 0)),
    out_shape=jax.ShapeDtypeStruct((HALF_V, 2 * EMBED), jnp.float32),
)


@functools.partial(
    pl.kernel,
    mesh=_mesh,
    out_type=jax.ShapeDtypeStruct((N, 2 * EMBED), jnp.float32),
    scratch_types=[
        pltpu.VMEM((NCH, CHUNK), jnp.int32),
        *([pltpu.VMEM((CHUNK, 2 * EMBED), jnp.float32)] * NBUF),
        *([pltpu.SemaphoreType.DMA] * NBUF),
    ],
)
def _gather(table_hbm, idx_hbm, out_hbm, idx_v, *bufs_and_sems):
    bufs = bufs_and_sems[:NBUF]
    sems = bufs_and_sems[NBUF:]
    wid = lax.axis_index("s") * NC + lax.axis_index("c")
    base = wid * PER_W
    pltpu.sync_copy(idx_hbm.at[wid], idx_v)

    for b in range(NBUF):
        pltpu.async_copy(table_hbm.at[idx_v.at[b]], bufs[b], sems[b])

    def body(g, carry):
        j0 = g * NBUF
        for b in range(NBUF):
            jj = j0 + b
            pltpu.make_async_copy(
                table_hbm.at[idx_v.at[jj]], bufs[b], sems[b]
            ).wait()
            pltpu.sync_copy(
                bufs[b], out_hbm.at[pl.ds(base + jj * CHUNK, CHUNK)]
            )
            nxt = jj + NBUF

            @pl.when(nxt < NCH)
            def _():
                pltpu.async_copy(table_hbm.at[idx_v.at[nxt]], bufs[b], sems[b])

        return carry

    lax.fori_loop(0, NCH // NBUF, body, 0)


def kernel(inputs, word_embeddings):
    flat = inputs.astype(jnp.int32).reshape(N)
    hi = flat >= HALF_V
    pair_idx = jnp.where(hi, flat - HALF_V, flat).reshape(NW, NCH, CHUNK)
    table2 = _repack(word_embeddings, word_embeddings)
    pairs = _gather(table2, pair_idx)
    out = jnp.where(hi[:, None], pairs[:, EMBED:], pairs[:, :EMBED])
    return out.reshape(BATCH, SEQ, EMBED)


# speedup vs baseline: 1.0039x; 1.0039x over previous
import functools

import jax
import jax.numpy as jnp
from jax import lax
from jax.experimental import pallas as pl
from jax.experimental.pallas import tpu as pltpu
from jax.experimental.pallas import tpu_sc as plsc

VOCAB_SIZE = 1000000
EMBED = 64
BATCH = 4096
SEQ = 50
N = BATCH * SEQ
NC = 2
NS = 16
NW = NC * NS
CHUNK = 128
PER_W = N // NW
NCH = PER_W // CHUNK
NBUF = 5

HALF_V = VOCAB_SIZE // 2
DEPAD_ROWS = 10000

_mesh = plsc.VectorSubcoreMesh(core_axis_name="c", subcore_axis_name="s")


def _repack_body(lo_ref, hi_ref, out_ref):
    out_ref[:, :EMBED] = lo_ref[...]
    out_ref[:, EMBED:] = hi_ref[...]


_repack = pl.pallas_call(
    _repack_body,
    grid=(HALF_V // DEPAD_ROWS,),
    in_specs=[
        pl.BlockSpec((DEPAD_ROWS, EMBED), lambda g: (g, 0)),
        pl.BlockSpec((DEPAD_ROWS, EMBED), lambda g: (g + HALF_V // DEPAD_ROWS, 0)),
    ],
    out_specs=pl.BlockSpec((DEPAD_ROWS, 2 * EMBED), lambda g: (g, 0)),
    out_shape=jax.ShapeDtypeStruct((HALF_V, 2 * EMBED), jnp.float32),
)


@functools.partial(
    pl.kernel,
    mesh=_mesh,
    out_type=jax.ShapeDtypeStruct((N, 2 * EMBED), jnp.float32),
    scratch_types=[
        pltpu.VMEM((NCH, CHUNK), jnp.int32),
        *([pltpu.VMEM((CHUNK, 2 * EMBED), jnp.float32)] * NBUF),
        *([pltpu.SemaphoreType.DMA] * NBUF),
    ],
)
def _gather(table_hbm, idx_hbm, out_hbm, idx_v, *bufs_and_sems):
    bufs = bufs_and_sems[:NBUF]
    sems = bufs_and_sems[NBUF:]
    wid = lax.axis_index("s") * NC + lax.axis_index("c")
    base = wid * PER_W
    pltpu.sync_copy(idx_hbm.at[wid], idx_v)

    for b in range(NBUF):
        pltpu.async_copy(table_hbm.at[idx_v.at[b]], bufs[b], sems[b])

    def body(g, carry):
        j0 = g * NBUF
        for b in range(NBUF):
            jj = j0 + b
            pltpu.make_async_copy(
                table_hbm.at[idx_v.at[jj]], bufs[b], sems[b]
            ).wait()
            pltpu.sync_copy(
                bufs[b], out_hbm.at[pl.ds(base + jj * CHUNK, CHUNK)]
            )
            nxt = jj + NBUF

            @pl.when(nxt < NCH)
            def _():
                pltpu.async_copy(table_hbm.at[idx_v.at[nxt]], bufs[b], sems[b])

        return carry

    lax.fori_loop(0, NCH // NBUF, body, 0)


def kernel(inputs, word_embeddings):
    flat = inputs.astype(jnp.int32).reshape(N)
    hi = flat >= HALF_V
    pair_idx = jnp.where(hi, flat - HALF_V, flat).reshape(NW, NCH, CHUNK)
    table2 = _repack(word_embeddings, word_embeddings)
    pairs = _gather(table2, pair_idx)
    out = jnp.where(hi[:, None], pairs[:, EMBED:], pairs[:, :EMBED])
    return out.reshape(BATCH, SEQ, EMBED)
